# baseline (speedup 1.0000x reference)
import functools

import jax
import jax.numpy as jnp
from jax import lax
from jax.experimental import pallas as pl
from jax.experimental.pallas import tpu as pltpu
from jax.experimental.pallas import tpu_sc as plsc


def _argmax_body(x_ref, idx_ref, idx2_ref, off_ref, *, bb, n_classes, d1,
                 table_rows):
    xb = x_ref[...]
    lanes = xb.shape[1]
    l = lax.broadcasted_iota(jnp.int32, (bb, lanes), 1)
    q = l // d1
    is_last = (l - q * d1) == (d1 - 1)
    masked = jnp.where(is_last, xb, jnp.float32(-jnp.inf))
    m = jnp.max(masked, axis=1, keepdims=True)
    big = jnp.int32(2147483647)
    cand = jnp.where(masked == m, q, big)
    y = jnp.min(cand, axis=1, keepdims=True)
    i = pl.program_id(0)
    row = i * bb + lax.broadcasted_iota(jnp.int32, (bb, 1), 0)
    idx = row * n_classes + y
    idx_ref[...] = idx
    s = idx * d1
    r0 = s >> 7
    r1 = jnp.minimum(r0 + 1, table_rows - 1)
    idx2_ref[...] = jnp.concatenate([r0, r1], axis=1)
    off_ref[...] = s & 127


def _expand_body(idx_ref, off_ref, g_ref, out_ref, *, bb, n_classes, d):
    pdim, ldim = out_ref.shape[1], out_ref.shape[2]
    i = pl.program_id(0)
    row = i * bb + lax.broadcasted_iota(jnp.int32, (bb, 1), 0)
    y = idx_ref[...] - row * n_classes
    off = off_ref[...]
    g = g_ref[...]
    w = lax.broadcasted_iota(jnp.int32, g.shape, 1)
    cols = [
        jnp.sum(jnp.where(w == off + f, g, jnp.float32(0.0)), axis=1,
                keepdims=True)
        for f in range(d)
    ]
    v128 = jnp.concatenate(cols * (ldim // d), axis=1)
    p = lax.broadcasted_iota(jnp.int32, (bb, pdim, ldim), 1)
    l = lax.broadcasted_iota(jnp.int32, (bb, pdim, ldim), 2)
    cls = p * (ldim // d) + l // d
    mask = cls == y.reshape(bb, 1, 1)
    out_ref[...] = jnp.where(mask, v128.reshape(bb, 1, ldim), jnp.float32(0.0))


def _make_sc_gather(n_rows, b_per_w, nc, ns):
    mesh = plsc.VectorSubcoreMesh(core_axis_name="c", subcore_axis_name="s")

    @functools.partial(
        pl.kernel,
        mesh=mesh,
        out_type=jax.ShapeDtypeStruct((2 * n_rows, 128), jnp.float32),
        scratch_types=[
            pltpu.VMEM((b_per_w,), jnp.int32),
            pltpu.VMEM((b_per_w,), jnp.int32),
            pltpu.VMEM((2 * b_per_w, 128), jnp.float32),
            pltpu.SemaphoreType.DMA,
        ],
    )
    def gather_k(table_hbm, idx2_hbm, out_hbm, idx_a, idx_b, buf, sem):
        wid = lax.axis_index("s") * nc + lax.axis_index("c")
        base = wid * 2 * b_per_w
        pltpu.sync_copy(idx2_hbm.at[pl.ds(base, b_per_w)], idx_a)
        pltpu.sync_copy(idx2_hbm.at[pl.ds(base + b_per_w, b_per_w)], idx_b)
        cp0 = pltpu.async_copy(table_hbm.at[idx_a], buf.at[pl.ds(0, b_per_w)],
                               sem)
        cp1 = pltpu.async_copy(table_hbm.at[idx_b],
                               buf.at[pl.ds(b_per_w, b_per_w)], sem)
        cp0.wait()
        cp1.wait()
        pltpu.sync_copy(buf, out_hbm.at[pl.ds(base, 2 * b_per_w)])

    return gather_k


def kernel(x):
    b, n_classes, d1 = x.shape
    d = d1 - 1
    bb = 128
    table_rows = (b * n_classes * d1) // 128
    x_flat = x.reshape(b, n_classes * d1)

    idx, idx2, off = pl.pallas_call(
        functools.partial(_argmax_body, bb=bb, n_classes=n_classes, d1=d1,
                          table_rows=table_rows),
        grid=(b // bb,),
        in_specs=[pl.BlockSpec((bb, n_classes * d1), lambda i: (i, 0))],
        out_specs=[
            pl.BlockSpec((bb, 1), lambda i: (i, 0)),
            pl.BlockSpec((bb, 2), lambda i: (i, 0)),
            pl.BlockSpec((bb, 1), lambda i: (i, 0)),
        ],
        out_shape=[
            jax.ShapeDtypeStruct((b, 1), jnp.int32),
            jax.ShapeDtypeStruct((b, 2), jnp.int32),
            jax.ShapeDtypeStruct((b, 1), jnp.int32),
        ],
    )(x_flat)

    info = plsc.get_sparse_core_info()
    nw = info.num_cores * info.num_subcores
    gather_k = _make_sc_gather(b, b // nw, info.num_cores, info.num_subcores)
    gathered = gather_k(x.reshape(table_rows, 128), idx2.reshape(2 * b))
    gathered = gathered.reshape(b, 256)

    pdim = (n_classes * d) // 128
    out3 = pl.pallas_call(
        functools.partial(_expand_body, bb=bb, n_classes=n_classes, d=d),
        grid=(b // bb,),
        in_specs=[
            pl.BlockSpec((bb, 1), lambda i: (i, 0)),
            pl.BlockSpec((bb, 1), lambda i: (i, 0)),
            pl.BlockSpec((bb, 256), lambda i: (i, 0)),
        ],
        out_specs=pl.BlockSpec((bb, pdim, 128), lambda i: (i, 0, 0)),
        out_shape=jax.ShapeDtypeStruct((b, pdim, 128), jnp.float32),
    )(idx, off, gathered)
    return out3.reshape(b, n_classes * d)


# speedup vs baseline: 10.8867x; 10.8867x over previous
import functools

import jax
import jax.numpy as jnp
from jax import lax
from jax.experimental import pallas as pl
from jax.experimental.pallas import tpu as pltpu
from jax.experimental.pallas import tpu_sc as plsc


def _argmax_body(x_ref, y_ref, idx_ref, *, lanes, n_classes, d, rows_per_f,
                 y_stride):
    xb = x_ref[0]
    c_iota = lax.broadcasted_iota(jnp.int32, xb.shape, 0)
    m = jnp.max(xb, axis=0, keepdims=True)
    big = jnp.int32(2147483647)
    cand = jnp.where(xb == m, c_iota, big)
    y = jnp.min(cand, axis=0, keepdims=True)
    y_t = jnp.transpose(y, (1, 0))
    y_ref[...] = y_t
    i = pl.program_id(0)
    b = i * lanes + lax.broadcasted_iota(jnp.int32, (lanes, 1), 0)
    f = lax.broadcasted_iota(jnp.int32, (1, d), 1)
    idx_ref[...] = f * rows_per_f + y_t * y_stride + (b >> 7)


def _expand_body(y_ref, g_ref, out_ref, *, bb, d):
    cols = out_ref.shape[1]
    y = y_ref[...]
    g = g_ref[...]
    bi = lax.broadcasted_iota(jnp.int32, g.shape, 0)
    li = lax.broadcasted_iota(jnp.int32, g.shape, 2)
    vals = jnp.sum(jnp.where(li == (bi & 127), g, jnp.float32(0.0)), axis=2)
    v128 = jnp.concatenate([vals] * (128 // d), axis=1)
    vfull = jnp.concatenate([v128] * (cols // 128), axis=1)
    j = lax.broadcasted_iota(jnp.int32, (bb, cols), 1)
    mask = (j >> 4) == y
    out_ref[...] = jnp.where(mask, vfull, jnp.float32(0.0))


def _make_sc_gather(n_idx, b_per_w, nc, ns):
    mesh = plsc.VectorSubcoreMesh(core_axis_name="c", subcore_axis_name="s")
    per_w = n_idx // (nc * ns)
    n_chunks = per_w // 128

    @functools.partial(
        pl.kernel,
        mesh=mesh,
        out_type=jax.ShapeDtypeStruct((n_idx, 128), jnp.float32),
        scratch_types=[
            pltpu.VMEM((per_w,), jnp.int32),
            pltpu.VMEM((128, 128), jnp.float32),
            pltpu.VMEM((128, 128), jnp.float32),
            pltpu.SemaphoreType.DMA,
            pltpu.SemaphoreType.DMA,
        ],
    )
    def gather_k(table_hbm, idx_hbm, out_hbm, idx_v, buf0, buf1, sem0, sem1):
        wid = lax.axis_index("s") * nc + lax.axis_index("c")
        base = wid * per_w
        pltpu.sync_copy(idx_hbm.at[pl.ds(base, per_w)], idx_v)
        bufs = (buf0, buf1)
        sems = (sem0, sem1)
        cps = [None, None]
        cps[0] = pltpu.async_copy(
            table_hbm.at[idx_v.at[pl.ds(0, 128)]], buf0, sem0)
        for k in range(n_chunks):
            if k + 1 < n_chunks:
                cps[(k + 1) % 2] = pltpu.async_copy(
                    table_hbm.at[idx_v.at[pl.ds((k + 1) * 128, 128)]],
                    bufs[(k + 1) % 2], sems[(k + 1) % 2])
            cps[k % 2].wait()
            pltpu.sync_copy(bufs[k % 2], out_hbm.at[pl.ds(base + k * 128, 128)])

    return gather_k


def kernel(x):
    b, n_classes, d1 = x.shape
    d = d1 - 1
    xt = jnp.transpose(x, (2, 1, 0))
    table_rows = (b * n_classes * d1) // 128
    rows_per_f = (n_classes * b) // 128

    lanes = 512
    y2, idx_list = pl.pallas_call(
        functools.partial(_argmax_body, lanes=lanes, n_classes=n_classes,
                          d=d, rows_per_f=rows_per_f, y_stride=b // 128),
        grid=(b // lanes,),
        in_specs=[pl.BlockSpec((1, n_classes, lanes), lambda i: (d, 0, i))],
        out_specs=[
            pl.BlockSpec((lanes, 1), lambda i: (i, 0)),
            pl.BlockSpec((lanes, d), lambda i: (i, 0)),
        ],
        out_shape=[
            jax.ShapeDtypeStruct((b, 1), jnp.int32),
            jax.ShapeDtypeStruct((b, d), jnp.int32),
        ],
    )(xt)

    info = plsc.get_sparse_core_info()
    nw = info.num_cores * info.num_subcores
    gather_k = _make_sc_gather(b * d, (b // nw) * d, info.num_cores,
                               info.num_subcores)
    g = gather_k(xt.reshape(table_rows, 128), idx_list.reshape(b * d))
    g3 = g.reshape(b, d, 128)

    bb = 128
    out = pl.pallas_call(
        functools.partial(_expand_body, bb=bb, d=d),
        grid=(b // bb,),
        in_specs=[
            pl.BlockSpec((bb, 1), lambda i: (i, 0)),
            pl.BlockSpec((bb, d, 128), lambda i: (i, 0, 0)),
        ],
        out_specs=pl.BlockSpec((bb, n_classes * d), lambda i: (i, 0)),
        out_shape=jax.ShapeDtypeStruct((b, n_classes * d), jnp.float32),
    )(y2, g3)
    return out


# speedup vs baseline: 33.1502x; 3.0450x over previous
import functools

import jax
import jax.numpy as jnp
from jax import lax
from jax.experimental import pallas as pl
from jax.experimental.pallas import tpu as pltpu
from jax.experimental.pallas import tpu_sc as plsc


def _argmax_body(x_ref, y_ref, idx_ref, *, lanes, n_classes, d, rows_per_f,
                 y_stride):
    xb = x_ref[0]
    c_iota = lax.broadcasted_iota(jnp.int32, xb.shape, 0)
    m = jnp.max(xb, axis=0, keepdims=True)
    big = jnp.int32(2147483647)
    cand = jnp.where(xb == m, c_iota, big)
    y = jnp.min(cand, axis=0, keepdims=True)
    y_t = jnp.transpose(y, (1, 0))
    y_ref[...] = y_t
    i = pl.program_id(0)
    b = i * lanes + lax.broadcasted_iota(jnp.int32, (lanes, 1), 0)
    f = lax.broadcasted_iota(jnp.int32, (1, d), 1)
    idx_ref[...] = (f * rows_per_f + (y_t >> 3) * (8 * y_stride)
                    + (b >> 7) * 8 + (y_t & 7))


def _expand_body(y_ref, g_ref, out_ref, *, bb, d):
    cols = out_ref.shape[1]
    y = y_ref[...]
    g = g_ref[...]
    bi = lax.broadcasted_iota(jnp.int32, g.shape, 0)
    li = lax.broadcasted_iota(jnp.int32, g.shape, 2)
    vals = jnp.sum(jnp.where(li == (bi & 127), g, jnp.float32(0.0)), axis=2)
    v128 = jnp.concatenate([vals] * (128 // d), axis=1)
    vfull = jnp.concatenate([v128] * (cols // 128), axis=1)
    j = lax.broadcasted_iota(jnp.int32, (bb, cols), 1)
    mask = (j >> 4) == y
    out_ref[...] = jnp.where(mask, vfull, jnp.float32(0.0))


def _make_sc_gather(n_idx, b_per_w, nc, ns):
    mesh = plsc.VectorSubcoreMesh(core_axis_name="c", subcore_axis_name="s")
    per_w = n_idx // (nc * ns)
    n_chunks = per_w // 128

    @functools.partial(
        pl.kernel,
        mesh=mesh,
        out_type=jax.ShapeDtypeStruct((n_idx, 128), jnp.float32),
        scratch_types=[
            pltpu.VMEM((per_w,), jnp.int32),
            pltpu.VMEM((128, 128), jnp.float32),
            pltpu.VMEM((128, 128), jnp.float32),
            pltpu.SemaphoreType.DMA,
            pltpu.SemaphoreType.DMA,
        ],
    )
    def gather_k(table_hbm, idx_hbm, out_hbm, idx_v, buf0, buf1, sem0, sem1):
        wid = lax.axis_index("s") * nc + lax.axis_index("c")
        base = wid * per_w
        pltpu.sync_copy(idx_hbm.at[pl.ds(base, per_w)], idx_v)
        bufs = (buf0, buf1)
        sems = (sem0, sem1)
        cps = [None, None]
        cps[0] = pltpu.async_copy(
            table_hbm.at[idx_v.at[pl.ds(0, 128)]], buf0, sem0)
        for k in range(n_chunks):
            if k + 1 < n_chunks:
                cps[(k + 1) % 2] = pltpu.async_copy(
                    table_hbm.at[idx_v.at[pl.ds((k + 1) * 128, 128)]],
                    bufs[(k + 1) % 2], sems[(k + 1) % 2])
            cps[k % 2].wait()
            pltpu.sync_copy(bufs[k % 2], out_hbm.at[pl.ds(base + k * 128, 128)])

    return gather_k


def kernel(x):
    b, n_classes, d1 = x.shape
    d = d1 - 1
    xt = jnp.transpose(x, (2, 1, 0))
    table_rows = (b * n_classes * d1) // 128
    rows_per_f = (n_classes * b) // 128

    lanes = 512
    y2, idx_list = pl.pallas_call(
        functools.partial(_argmax_body, lanes=lanes, n_classes=n_classes,
                          d=d, rows_per_f=rows_per_f, y_stride=b // 128),
        grid=(b // lanes,),
        in_specs=[pl.BlockSpec((1, n_classes, lanes), lambda i: (d, 0, i))],
        out_specs=[
            pl.BlockSpec((lanes, 1), lambda i: (i, 0)),
            pl.BlockSpec((lanes, d), lambda i: (i, 0)),
        ],
        out_shape=[
            jax.ShapeDtypeStruct((b, 1), jnp.int32),
            jax.ShapeDtypeStruct((b, d), jnp.int32),
        ],
    )(xt)

    info = plsc.get_sparse_core_info()
    nw = info.num_cores * info.num_subcores
    gather_k = _make_sc_gather(b * d, (b // nw) * d, info.num_cores,
                               info.num_subcores)
    table = (xt.reshape(d1, n_classes // 8, 8, b // 128, 128)
             .transpose(0, 1, 3, 2, 4)
             .reshape(table_rows, 128))
    g = gather_k(table, idx_list.reshape(b * d))
    g3 = g.reshape(b, d, 128)

    bb = 128
    out = pl.pallas_call(
        functools.partial(_expand_body, bb=bb, d=d),
        grid=(b // bb,),
        in_specs=[
            pl.BlockSpec((bb, 1), lambda i: (i, 0)),
            pl.BlockSpec((bb, d, 128), lambda i: (i, 0, 0)),
        ],
        out_specs=pl.BlockSpec((bb, n_classes * d), lambda i: (i, 0)),
        out_shape=jax.ShapeDtypeStruct((b, n_classes * d), jnp.float32),
    )(y2, g3)
    return out


# speedup vs baseline: 33.4815x; 1.0100x over previous
import functools

import jax
import jax.numpy as jnp
from jax import lax
from jax.experimental import pallas as pl
from jax.experimental.pallas import tpu as pltpu
from jax.experimental.pallas import tpu_sc as plsc


def _argmax_body(x_ref, y_ref, idx_ref, *, lanes, n_classes, d, rows_per_f,
                 y_stride):
    xb = x_ref[0]
    c_iota = lax.broadcasted_iota(jnp.int32, xb.shape, 0)
    m = jnp.max(xb, axis=0, keepdims=True)
    big = jnp.int32(2147483647)
    cand = jnp.where(xb == m, c_iota, big)
    y = jnp.min(cand, axis=0, keepdims=True)
    y_t = jnp.transpose(y, (1, 0))
    y_ref[...] = y_t
    i = pl.program_id(0)
    b = i * lanes + lax.broadcasted_iota(jnp.int32, (lanes, 1), 0)
    f = lax.broadcasted_iota(jnp.int32, (1, d), 1)
    idx_ref[...] = (f * rows_per_f + (y_t >> 3) * (8 * y_stride)
                    + (b >> 7) * 8 + (y_t & 7))


def _expand_body(y_ref, g_ref, out_ref, *, bb, d):
    cols = out_ref.shape[1]
    y = y_ref[...]
    g = g_ref[...]
    bi = lax.broadcasted_iota(jnp.int32, g.shape, 0)
    li = lax.broadcasted_iota(jnp.int32, g.shape, 2)
    vals = jnp.sum(jnp.where(li == (bi & 127), g, jnp.float32(0.0)), axis=2)
    v128 = jnp.concatenate([vals] * (128 // d), axis=1)
    vfull = jnp.concatenate([v128] * (cols // 128), axis=1)
    j = lax.broadcasted_iota(jnp.int32, (bb, cols), 1)
    mask = (j >> 4) == y
    out_ref[...] = jnp.where(mask, vfull, jnp.float32(0.0))


def _make_sc_gather(n_idx, b_per_w, nc, ns):
    mesh = plsc.VectorSubcoreMesh(core_axis_name="c", subcore_axis_name="s")
    per_w = n_idx // (nc * ns)
    n_chunks = per_w // 128

    @functools.partial(
        pl.kernel,
        mesh=mesh,
        out_type=jax.ShapeDtypeStruct((n_idx, 128), jnp.float32),
        scratch_types=[
            pltpu.VMEM((per_w,), jnp.int32),
            pltpu.VMEM((128, 128), jnp.float32),
            pltpu.VMEM((128, 128), jnp.float32),
            pltpu.SemaphoreType.DMA,
            pltpu.SemaphoreType.DMA,
        ],
    )
    def gather_k(table_hbm, idx_hbm, out_hbm, idx_v, buf0, buf1, sem0, sem1):
        wid = lax.axis_index("s") * nc + lax.axis_index("c")
        base = wid * per_w
        pltpu.sync_copy(idx_hbm.at[pl.ds(base, per_w)], idx_v)
        bufs = (buf0, buf1)
        sems = (sem0, sem1)
        cps = [None, None]
        cps[0] = pltpu.async_copy(
            table_hbm.at[idx_v.at[pl.ds(0, 128)]], buf0, sem0)
        for k in range(n_chunks):
            if k + 1 < n_chunks:
                cps[(k + 1) % 2] = pltpu.async_copy(
                    table_hbm.at[idx_v.at[pl.ds((k + 1) * 128, 128)]],
                    bufs[(k + 1) % 2], sems[(k + 1) % 2])
            cps[k % 2].wait()
            pltpu.sync_copy(bufs[k % 2], out_hbm.at[pl.ds(base + k * 128, 128)])

    return gather_k


def kernel(x):
    b, n_classes, d1 = x.shape
    d = d1 - 1
    xt = jnp.transpose(x, (2, 1, 0))
    table_rows = (b * n_classes * d1) // 128
    rows_per_f = (n_classes * b) // 128

    lanes = 512
    y2, idx_list = pl.pallas_call(
        functools.partial(_argmax_body, lanes=lanes, n_classes=n_classes,
                          d=d, rows_per_f=rows_per_f, y_stride=b // 128),
        grid=(b // lanes,),
        in_specs=[pl.BlockSpec((1, n_classes, lanes), lambda i: (d, 0, i))],
        out_specs=[
            pl.BlockSpec((lanes, 1), lambda i: (i, 0)),
            pl.BlockSpec((lanes, d), lambda i: (i, 0)),
        ],
        out_shape=[
            jax.ShapeDtypeStruct((b, 1), jnp.int32),
            jax.ShapeDtypeStruct((b, d), jnp.int32),
        ],
    )(xt)

    info = plsc.get_sparse_core_info()
    nw = info.num_cores * info.num_subcores
    gather_k = _make_sc_gather(b * d, (b // nw) * d, info.num_cores,
                               info.num_subcores)
    table = (xt.reshape(d1, n_classes // 8, 8, b // 128, 128)
             .transpose(0, 1, 3, 2, 4)
             .reshape(table_rows, 128))
    g = gather_k(table, idx_list.reshape(b * d))
    g3 = g.reshape(b, d, 128)

    bb = 256
    out = pl.pallas_call(
        functools.partial(_expand_body, bb=bb, d=d),
        grid=(b // bb,),
        in_specs=[
            pl.BlockSpec((bb, 1), lambda i: (i, 0)),
            pl.BlockSpec((bb, d, 128), lambda i: (i, 0, 0)),
        ],
        out_specs=pl.BlockSpec((bb, n_classes * d), lambda i: (i, 0)),
        out_shape=jax.ShapeDtypeStruct((b, n_classes * d), jnp.float32),
    )(y2, g3)
    return out
